# baseline (speedup 1.0000x reference)
import jax
import jax.numpy as jnp
from jax import lax
from jax.experimental import pallas as pl
from jax.experimental.pallas import tpu as pltpu
from jax.experimental.pallas import tpu_sc as plsc

_H = 224
_W = 224
_NPAIR = 14
_CH = 32
_EYW = 240
_NW = _W // 16


def _body(img, out, inb, outb, eyb, oyb, in_s0, in_s1, out_s0, out_s1):
    batch = lax.axis_index("c")
    seg = lax.axis_index("s")
    k0 = seg * _NPAIR
    in_sems = (in_s0, in_s1)
    out_sems = (out_s0, out_s1)
    lanes = lax.iota(jnp.int32, 16)

    def issue_in(b, hc, k):
        rp = jnp.maximum(k - 1, 0)
        rn = jnp.minimum(k + 1, _H - 1)
        for j, r in enumerate((rp, k, rn)):
            pltpu.async_copy(
                img.at[batch, r, pl.ds(hc * _CH, _CH), :],
                inb.at[b, j],
                in_sems[b],
            )

    def wait_in(b, hc):
        for j in range(3):
            pltpu.make_async_copy(
                img.at[0, 0, pl.ds(hc * _CH, _CH), :],
                inb.at[b, j],
                in_sems[b],
            ).wait()

    def issue_out(b, hc, k):
        for r in range(2):
            pltpu.async_copy(
                outb.at[b, r],
                out.at[batch, 2 * k + r, pl.ds(hc * _CH, _CH), :],
                out_sems[b],
            )

    def wait_out(b, hc):
        for r in range(2):
            pltpu.make_async_copy(
                outb.at[b, r],
                out.at[0, 0, pl.ds(hc * _CH, _CH), :],
                out_sems[b],
            ).wait()

    def compute(b):
        def a_ch(ch, _):
            def a_w(w16, _):
                w0 = w16 * 16
                src = pl.ds(w0, 16)
                pc = inb[b, 0, ch, src]
                cc = inb[b, 1, ch, src]
                nc = inb[b, 2, ch, src]
                dst = pl.ds(ch * _EYW + w0 + 16, 16)
                eyb[dst] = 0.25 * pc + 0.75 * cc
                oyb[dst] = 0.75 * cc + 0.25 * nc
                return 0
            lax.fori_loop(0, _NW, a_w, 0, unroll=7)
            return 0
        lax.fori_loop(0, _CH, a_ch, 0)

        ch16 = lanes * _EYW
        for grp in range(2):
            base = grp * 16 * _EYW
            for ref in (eyb, oyb):
                v0 = plsc.load_gather(ref, [ch16 + (base + 16)])
                plsc.store_scatter(ref, [ch16 + (base + 15)], v0)
                v1 = plsc.load_gather(ref, [ch16 + (base + 239)])
                plsc.store_scatter(ref, [ch16 + (base + 240)], v1)

        ebuf = outb.at[b, 0]
        obuf = outb.at[b, 1]

        def b_ch(ch, _):
            chv = jnp.full((16,), 0, jnp.int32) + ch

            def b_w(w16, _):
                w0 = w16 * 16
                b0 = ch * _EYW + w0 + 16
                ey_m1 = plsc.load_gather(eyb, [lanes + (b0 - 1)])
                ey_0 = eyb[pl.ds(b0, 16)]
                ey_p1 = plsc.load_gather(eyb, [lanes + (b0 + 1)])
                oy_m1 = plsc.load_gather(oyb, [lanes + (b0 - 1)])
                oy_0 = oyb[pl.ds(b0, 16)]
                oy_p1 = plsc.load_gather(oyb, [lanes + (b0 + 1)])
                wev = 2 * lanes + 2 * w0
                wod = wev + 1
                plsc.store_scatter(ebuf, [chv, wev],
                                   0.25 * ey_m1 + 0.75 * ey_0)
                plsc.store_scatter(ebuf, [chv, wod],
                                   0.75 * ey_0 + 0.25 * ey_p1)
                plsc.store_scatter(obuf, [chv, wev],
                                   0.25 * oy_m1 + 0.75 * oy_0)
                plsc.store_scatter(obuf, [chv, wod],
                                   0.75 * oy_0 + 0.25 * oy_p1)
                return 0
            lax.fori_loop(0, _NW, b_w, 0, unroll=7)
            return 0
        lax.fori_loop(0, _CH, b_ch, 0)

    def do_pair(b, hc, k, prefetch_k, do_wait_out):
        wait_in(b, hc)
        if prefetch_k is not None:
            issue_in(1 - b, hc, prefetch_k)
        if do_wait_out:
            wait_out(b, hc)
        compute(b)
        issue_out(b, hc, k)

    for hc in range(2):
        issue_in(0, hc, k0)
        do_pair(0, hc, k0, k0 + 1, False)
        do_pair(1, hc, k0 + 1, k0 + 2, False)

        def loop(i, _):
            do_pair(0, hc, k0 + 2 * i, k0 + 2 * i + 1, True)
            do_pair(1, hc, k0 + 2 * i + 1, k0 + 2 * i + 2, True)
            return 0

        lax.fori_loop(1, _NPAIR // 2 - 1, loop, 0)
        do_pair(0, hc, k0 + _NPAIR - 2, k0 + _NPAIR - 1, True)
        do_pair(1, hc, k0 + _NPAIR - 1, None, True)
        wait_out(0, hc)
        wait_out(1, hc)


def kernel(img):
    n, h, w, ch = img.shape
    img_t = jnp.transpose(img, (0, 1, 3, 2))
    run = pl.kernel(
        _body,
        out_type=jax.ShapeDtypeStruct((n, 2 * h, ch, 2 * w), jnp.float32),
        mesh=plsc.VectorSubcoreMesh(
            core_axis_name="c", subcore_axis_name="s",
            num_cores=2, num_subcores=16,
        ),
        compiler_params=pltpu.CompilerParams(
            use_tc_tiling_on_sc=True, needs_layout_passes=False,
        ),
        scratch_types=[
            pltpu.VMEM((2, 3, _CH, _W), jnp.float32),
            pltpu.VMEM((2, 2, _CH, 2 * _W), jnp.float32),
            pltpu.VMEM((_CH * _EYW + 16,), jnp.float32),
            pltpu.VMEM((_CH * _EYW + 16,), jnp.float32),
            pltpu.SemaphoreType.DMA,
            pltpu.SemaphoreType.DMA,
            pltpu.SemaphoreType.DMA,
            pltpu.SemaphoreType.DMA,
        ],
    )
    out_t = run(img_t)
    return jnp.transpose(out_t, (0, 1, 3, 2))


# speedup vs baseline: 1.0071x; 1.0071x over previous
import jax
import jax.numpy as jnp
from jax import lax
from jax.experimental import pallas as pl
from jax.experimental.pallas import tpu as pltpu
from jax.experimental.pallas import tpu_sc as plsc

_H = 224
_W = 224
_NPAIR = 14
_CH = 32
_EYW = 240
_NW = _W // 16


def _body(img, out, inb, outb, eyb, oyb, in_s0, in_s1, out_s0, out_s1):
    batch = lax.axis_index("c")
    seg = lax.axis_index("s")
    k0 = seg * _NPAIR
    in_sems = (in_s0, in_s1)
    out_sems = (out_s0, out_s1)
    lanes = lax.iota(jnp.int32, 16)

    def issue_in(b, hc, k):
        rp = jnp.maximum(k - 1, 0)
        rn = jnp.minimum(k + 1, _H - 1)
        for j, r in enumerate((rp, k, rn)):
            pltpu.async_copy(
                img.at[batch, r, pl.ds(hc * _CH, _CH), :],
                inb.at[b, j],
                in_sems[b],
            )

    def wait_in(b, hc):
        for j in range(3):
            pltpu.make_async_copy(
                img.at[0, 0, pl.ds(hc * _CH, _CH), :],
                inb.at[b, j],
                in_sems[b],
            ).wait()

    def issue_out(b, hc, k):
        for r in range(2):
            pltpu.async_copy(
                outb.at[b, r],
                out.at[batch, 2 * k + r, pl.ds(hc * _CH, _CH), :],
                out_sems[b],
            )

    def wait_out(b, hc):
        for r in range(2):
            pltpu.make_async_copy(
                outb.at[b, r],
                out.at[0, 0, pl.ds(hc * _CH, _CH), :],
                out_sems[b],
            ).wait()

    def compute(b):
        def a_ch(ch, _):
            def a_w(w16, _):
                w0 = w16 * 16
                src = pl.ds(w0, 16)
                pc = inb[b, 0, ch, src]
                cc = inb[b, 1, ch, src]
                nc = inb[b, 2, ch, src]
                dst = pl.ds(ch * _EYW + w0 + 16, 16)
                eyb[dst] = 0.25 * pc + 0.75 * cc
                oyb[dst] = 0.75 * cc + 0.25 * nc
                return 0
            lax.fori_loop(0, _NW, a_w, 0, unroll=2)
            return 0
        lax.fori_loop(0, _CH, a_ch, 0)

        ch16 = lanes * _EYW
        for grp in range(2):
            base = grp * 16 * _EYW
            for ref in (eyb, oyb):
                v0 = plsc.load_gather(ref, [ch16 + (base + 16)])
                plsc.store_scatter(ref, [ch16 + (base + 15)], v0)
                v1 = plsc.load_gather(ref, [ch16 + (base + 239)])
                plsc.store_scatter(ref, [ch16 + (base + 240)], v1)

        ebuf = outb.at[b, 0]
        obuf = outb.at[b, 1]

        def b_ch(ch, _):
            chv = jnp.full((16,), 0, jnp.int32) + ch
            base0 = ch * _EYW + 16

            def b_w(w16, carry):
                im1, ip1, wev = carry
                b0 = base0 + w16 * 16
                ey_m1 = plsc.load_gather(eyb, [im1])
                ey_0 = eyb[pl.ds(b0, 16)]
                ey_p1 = plsc.load_gather(eyb, [ip1])
                oy_m1 = plsc.load_gather(oyb, [im1])
                oy_0 = oyb[pl.ds(b0, 16)]
                oy_p1 = plsc.load_gather(oyb, [ip1])
                wod = wev + 1
                plsc.store_scatter(ebuf, [chv, wev],
                                   0.25 * ey_m1 + 0.75 * ey_0)
                plsc.store_scatter(ebuf, [chv, wod],
                                   0.75 * ey_0 + 0.25 * ey_p1)
                plsc.store_scatter(obuf, [chv, wev],
                                   0.25 * oy_m1 + 0.75 * oy_0)
                plsc.store_scatter(obuf, [chv, wod],
                                   0.75 * oy_0 + 0.25 * oy_p1)
                return im1 + 16, ip1 + 16, wev + 32
            lax.fori_loop(
                0, _NW, b_w,
                (lanes + (base0 - 1), lanes + (base0 + 1), 2 * lanes),
                unroll=2,
            )
            return 0
        lax.fori_loop(0, _CH, b_ch, 0)

    def do_pair(b, hc, k, prefetch_k, do_wait_out):
        wait_in(b, hc)
        if prefetch_k is not None:
            issue_in(1 - b, hc, prefetch_k)
        if do_wait_out:
            wait_out(b, hc)
        compute(b)
        issue_out(b, hc, k)

    for hc in range(2):
        issue_in(0, hc, k0)
        do_pair(0, hc, k0, k0 + 1, False)
        do_pair(1, hc, k0 + 1, k0 + 2, False)

        def loop(i, _):
            do_pair(0, hc, k0 + 2 * i, k0 + 2 * i + 1, True)
            do_pair(1, hc, k0 + 2 * i + 1, k0 + 2 * i + 2, True)
            return 0

        lax.fori_loop(1, _NPAIR // 2 - 1, loop, 0)
        do_pair(0, hc, k0 + _NPAIR - 2, k0 + _NPAIR - 1, True)
        do_pair(1, hc, k0 + _NPAIR - 1, None, True)
        wait_out(0, hc)
        wait_out(1, hc)


def kernel(img):
    n, h, w, ch = img.shape
    img_t = jnp.transpose(img, (0, 1, 3, 2))
    run = pl.kernel(
        _body,
        out_type=jax.ShapeDtypeStruct((n, 2 * h, ch, 2 * w), jnp.float32),
        mesh=plsc.VectorSubcoreMesh(
            core_axis_name="c", subcore_axis_name="s",
            num_cores=2, num_subcores=16,
        ),
        compiler_params=pltpu.CompilerParams(
            use_tc_tiling_on_sc=True, needs_layout_passes=False,
        ),
        scratch_types=[
            pltpu.VMEM((2, 3, _CH, _W), jnp.float32),
            pltpu.VMEM((2, 2, _CH, 2 * _W), jnp.float32),
            pltpu.VMEM((_CH * _EYW + 16,), jnp.float32),
            pltpu.VMEM((_CH * _EYW + 16,), jnp.float32),
            pltpu.SemaphoreType.DMA,
            pltpu.SemaphoreType.DMA,
            pltpu.SemaphoreType.DMA,
            pltpu.SemaphoreType.DMA,
        ],
    )
    out_t = run(img_t)
    return jnp.transpose(out_t, (0, 1, 3, 2))


# speedup vs baseline: 1.0074x; 1.0003x over previous
import jax
import jax.numpy as jnp
from jax import lax
from jax.experimental import pallas as pl
from jax.experimental.pallas import tpu as pltpu
from jax.experimental.pallas import tpu_sc as plsc

_H = 224
_W = 224
_NPAIR = 14
_CH = 32
_EYW = 240
_NW = _W // 16


def _body(img, out, inb, outb, eyb, oyb, in_s0, in_s1, out_s0, out_s1):
    batch = lax.axis_index("c")
    seg = lax.axis_index("s")
    k0 = seg * _NPAIR
    in_sems = (in_s0, in_s1)
    out_sems = (out_s0, out_s1)
    lanes = lax.iota(jnp.int32, 16)

    def issue_in(b, hc, k):
        rp = jnp.maximum(k - 1, 0)
        rn = jnp.minimum(k + 1, _H - 1)
        for j, r in enumerate((rp, k, rn)):
            pltpu.async_copy(
                img.at[batch, r, pl.ds(hc * _CH, _CH), :],
                inb.at[b, j],
                in_sems[b],
            )

    def wait_in(b, hc):
        for j in range(3):
            pltpu.make_async_copy(
                img.at[0, 0, pl.ds(hc * _CH, _CH), :],
                inb.at[b, j],
                in_sems[b],
            ).wait()

    def issue_out(b, hc, k):
        for r in range(2):
            pltpu.async_copy(
                outb.at[b, r],
                out.at[batch, 2 * k + r, pl.ds(hc * _CH, _CH), :],
                out_sems[b],
            )

    def wait_out(b, hc):
        for r in range(2):
            pltpu.make_async_copy(
                outb.at[b, r],
                out.at[0, 0, pl.ds(hc * _CH, _CH), :],
                out_sems[b],
            ).wait()

    def compute(b):
        def a_ch(ch, _):
            def a_w(w16, _):
                w0 = w16 * 16
                src = pl.ds(w0, 16)
                pc = inb[b, 0, ch, src]
                cc = inb[b, 1, ch, src]
                nc = inb[b, 2, ch, src]
                dst = pl.ds(ch * _EYW + w0 + 16, 16)
                eyb[dst] = 0.25 * pc + 0.75 * cc
                oyb[dst] = 0.75 * cc + 0.25 * nc
                return 0
            lax.fori_loop(0, _NW, a_w, 0, unroll=2)
            return 0
        lax.fori_loop(0, _CH, a_ch, 0)

        ch16 = lanes * _EYW
        for grp in range(2):
            base = grp * 16 * _EYW
            for ref in (eyb, oyb):
                v0 = plsc.load_gather(ref, [ch16 + (base + 16)])
                plsc.store_scatter(ref, [ch16 + (base + 15)], v0)
                v1 = plsc.load_gather(ref, [ch16 + (base + 239)])
                plsc.store_scatter(ref, [ch16 + (base + 240)], v1)

        ebuf = outb.at[b, 0]
        obuf = outb.at[b, 1]

        def b_ch(ch, _):
            chv = jnp.full((16,), 0, jnp.int32) + ch

            def b_w(w16, _):
                w0 = w16 * 16
                b0 = ch * _EYW + w0 + 16
                ey_m1 = plsc.load_gather(eyb, [lanes + (b0 - 1)])
                ey_0 = eyb[pl.ds(b0, 16)]
                ey_p1 = plsc.load_gather(eyb, [lanes + (b0 + 1)])
                oy_m1 = plsc.load_gather(oyb, [lanes + (b0 - 1)])
                oy_0 = oyb[pl.ds(b0, 16)]
                oy_p1 = plsc.load_gather(oyb, [lanes + (b0 + 1)])
                wev = 2 * lanes + 2 * w0
                wod = wev + 1
                plsc.store_scatter(ebuf, [chv, wev],
                                   0.25 * ey_m1 + 0.75 * ey_0)
                plsc.store_scatter(ebuf, [chv, wod],
                                   0.75 * ey_0 + 0.25 * ey_p1)
                plsc.store_scatter(obuf, [chv, wev],
                                   0.25 * oy_m1 + 0.75 * oy_0)
                plsc.store_scatter(obuf, [chv, wod],
                                   0.75 * oy_0 + 0.25 * oy_p1)
                return 0
            lax.fori_loop(0, _NW, b_w, 0, unroll=2)
            return 0
        lax.fori_loop(0, _CH, b_ch, 0)

    def do_pair(b, hc, k, prefetch_k, do_wait_out):
        wait_in(b, hc)
        if prefetch_k is not None:
            issue_in(1 - b, hc, prefetch_k)
        if do_wait_out:
            wait_out(b, hc)
        compute(b)
        issue_out(b, hc, k)

    for hc in range(2):
        issue_in(0, hc, k0)
        do_pair(0, hc, k0, k0 + 1, False)
        do_pair(1, hc, k0 + 1, k0 + 2, False)

        def loop(i, _):
            do_pair(0, hc, k0 + 2 * i, k0 + 2 * i + 1, True)
            do_pair(1, hc, k0 + 2 * i + 1, k0 + 2 * i + 2, True)
            return 0

        lax.fori_loop(1, _NPAIR // 2 - 1, loop, 0)
        do_pair(0, hc, k0 + _NPAIR - 2, k0 + _NPAIR - 1, True)
        do_pair(1, hc, k0 + _NPAIR - 1, None, True)
        wait_out(0, hc)
        wait_out(1, hc)


def kernel(img):
    n, h, w, ch = img.shape
    img_t = jnp.transpose(img, (0, 1, 3, 2))
    run = pl.kernel(
        _body,
        out_type=jax.ShapeDtypeStruct((n, 2 * h, ch, 2 * w), jnp.float32),
        mesh=plsc.VectorSubcoreMesh(
            core_axis_name="c", subcore_axis_name="s",
            num_cores=2, num_subcores=16,
        ),
        compiler_params=pltpu.CompilerParams(
            use_tc_tiling_on_sc=True, needs_layout_passes=False,
        ),
        scratch_types=[
            pltpu.VMEM((2, 3, _CH, _W), jnp.float32),
            pltpu.VMEM((2, 2, _CH, 2 * _W), jnp.float32),
            pltpu.VMEM((_CH * _EYW + 16,), jnp.float32),
            pltpu.VMEM((_CH * _EYW + 16,), jnp.float32),
            pltpu.SemaphoreType.DMA,
            pltpu.SemaphoreType.DMA,
            pltpu.SemaphoreType.DMA,
            pltpu.SemaphoreType.DMA,
        ],
    )
    out_t = run(img_t)
    return jnp.transpose(out_t, (0, 1, 3, 2))
